# baseline (speedup 1.0000x reference)
import jax
import jax.numpy as jnp
from jax.experimental import pallas as pl
from jax.experimental.pallas import tpu as pltpu

N_ROWS = 64
C = 256
_HW = (56 * 56, 28 * 28, 14 * 14, 7 * 7, 4 * 4)


def _make_scatter_body(li, hw):
    def body(lids, chs, ain, aout, fill, sem):
        del ain
        fill[...] = jnp.full((hw,), -1000.0, jnp.float32)
        for i in range(N_ROWS):
            lid = lids[i]
            c = chs[i]

            @pl.when(lid == li)
            def _(i=i, c=c):
                pltpu.make_async_copy(fill, aout.at[i * C + c], sem).start()
        for i in range(N_ROWS):
            lid = lids[i]
            c = chs[i]

            @pl.when(lid == li)
            def _(i=i, c=c):
                pltpu.make_async_copy(fill, aout.at[i * C + c], sem).wait()

    return body


def _ablate_level(li, a, layer_ids, ch):
    hw = _HW[li]
    flat = a.reshape(N_ROWS * C, hw)
    out = pl.pallas_call(
        _make_scatter_body(li, hw),
        in_specs=[pl.BlockSpec(memory_space=pltpu.SMEM),
                  pl.BlockSpec(memory_space=pltpu.SMEM),
                  pl.BlockSpec(memory_space=pl.ANY)],
        out_specs=pl.BlockSpec(memory_space=pl.ANY),
        out_shape=jax.ShapeDtypeStruct(flat.shape, flat.dtype),
        input_output_aliases={2: 0},
        scratch_shapes=[pltpu.VMEM((hw,), jnp.float32),
                        pltpu.SemaphoreType.DMA],
    )(layer_ids, ch, flat)
    return out.reshape(a.shape)


def kernel(act_0, act_1, act_2, act_3, act_pool, indices, x):
    del x
    acts = (act_0, act_1, act_2, act_3, act_pool)
    layer_ids = (indices // C).astype(jnp.int32)
    ch = (indices % C).astype(jnp.int32)
    return tuple(
        _ablate_level(li, a, layer_ids, ch) for li, a in enumerate(acts)
    )


# speedup vs baseline: 1.6654x; 1.6654x over previous
import jax
import jax.numpy as jnp
from jax.experimental import pallas as pl
from jax.experimental.pallas import tpu as pltpu

N_ROWS = 64
C = 256
_HW = (56 * 56, 28 * 28, 14 * 14, 7 * 7, 4 * 4)
_RPB = (1, 2, 8, 16, 16)
_STEPS = tuple(N_ROWS // r for r in _RPB)
_BASE = (0, 64, 96, 104, 108)
_TOTAL = 112


def _body(lids, chs, i0, i1, i2, i3, i4, o0, o1, o2, o3, o4):
    s = pl.program_id(0)
    ins = (i0, i1, i2, i3, i4)
    outs = (o0, o1, o2, o3, o4)
    for li in range(5):
        base = _BASE[li]
        rpb = _RPB[li]
        hw = _HW[li]

        @pl.when((s >= base) & (s < base + _STEPS[li]))
        def _(li=li, base=base, rpb=rpb, hw=hw):
            ain = ins[li]
            aout = outs[li]
            aout[...] = ain[...]
            b = s - base
            for r in range(rpb):
                i = b * rpb + r
                lid = lids[i]
                c = chs[i]

                @pl.when(lid == li)
                def _(r=r, c=c, hw=hw, aout=aout):
                    aout[r, c, :] = jnp.full((hw,), -1000.0, jnp.float32)

    return


def kernel(act_0, act_1, act_2, act_3, act_pool, indices, x):
    del x
    acts = (act_0, act_1, act_2, act_3, act_pool)
    layer_ids = (indices // C).astype(jnp.int32)
    ch = (indices % C).astype(jnp.int32)
    flats = [a.reshape(N_ROWS, C, hw) for a, hw in zip(acts, _HW)]

    def make_spec(li):
        rpb = _RPB[li]
        base = _BASE[li]
        nsteps = _STEPS[li]

        def index_map(s, lids, chs):
            b = jnp.clip(s - base, 0, nsteps - 1)
            return (b, 0, 0)

        return pl.BlockSpec((rpb, C, _HW[li]), index_map)

    grid_spec = pltpu.PrefetchScalarGridSpec(
        num_scalar_prefetch=2,
        grid=(_TOTAL,),
        in_specs=[make_spec(li) for li in range(5)],
        out_specs=[make_spec(li) for li in range(5)],
    )
    outs = pl.pallas_call(
        _body,
        grid_spec=grid_spec,
        out_shape=[jax.ShapeDtypeStruct(f.shape, f.dtype) for f in flats],
    )(layer_ids, ch, *flats)
    return tuple(o.reshape(a.shape) for o, a in zip(outs, acts))
